# baseline (speedup 1.0000x reference)
import jax
import jax.numpy as jnp
from jax.experimental import pallas as pl
from jax.experimental.pallas import tpu as pltpu


def _round_up(x, m):
    return (x + m - 1) // m * m


def _fused_kernel(ft_ref, w01_ref, w02a_ref, w02b_ref, w12_ref,
                  wf_ref, bias_ref, out_ref):
    D = w02a_ref.shape[0]
    f0 = ft_ref[0]
    f1 = ft_ref[1]
    f2 = ft_ref[2]

    r01 = jnp.dot(f0, w01_ref[:D], preferred_element_type=jnp.float32)
    r01 = r01 + jnp.dot(f1, w01_ref[D:], preferred_element_type=jnp.float32)
    r12 = jnp.dot(f1, w12_ref[:D], preferred_element_type=jnp.float32)
    r12 = r12 + jnp.dot(f2, w12_ref[D:], preferred_element_type=jnp.float32)
    r02 = jnp.dot(f0, w02a_ref[...], preferred_element_type=jnp.float32)
    r02 = r02 + jnp.dot(f2, w02b_ref[...],
                        preferred_element_type=jnp.float32)
    r01 = r01.astype(jnp.bfloat16)
    r02 = r02.astype(jnp.bfloat16)
    r12 = r12.astype(jnp.bfloat16)

    wf = wf_ref[...]
    out = jnp.dot(r01, wf[:D], preferred_element_type=jnp.float32)
    out = out + jnp.dot(r02, wf[D:2 * D], preferred_element_type=jnp.float32)
    out = out + jnp.dot(r12, wf[2 * D:], preferred_element_type=jnp.float32)
    out_ref[...] = out + bias_ref[...]


def kernel(features, wd, bd, wf, bf):
    B, three, D = features.shape
    NF = wf.shape[1]

    w01 = wd[:2 * D, :D].astype(jnp.bfloat16)
    w12 = wd[D:, 2 * D:].astype(jnp.bfloat16)
    w02a = wd[:D, D:2 * D].astype(jnp.bfloat16)
    w02b = wd[2 * D:, D:2 * D].astype(jnp.bfloat16)
    bias2 = jnp.dot(bd, wf, preferred_element_type=jnp.float32) + bf
    wf = wf.astype(jnp.bfloat16)

    ft = features.transpose((1, 0, 2)).astype(jnp.bfloat16)

    TB = 1024
    B_pad = _round_up(B, 2 * TB)
    if B_pad != B:
        ft = jnp.pad(ft, ((0, 0), (0, B_pad - B), (0, 0)))
    n_inner = B_pad // TB // 2

    compiler_params = pltpu.CompilerParams(
        dimension_semantics=("parallel", "arbitrary"),
        vmem_limit_bytes=64 * 1024 * 1024,
    )

    def _tile3(c, j, n=n_inner):
        return (0, c * n + j, 0)

    def _tile(c, j, n=n_inner):
        return (c * n + j, 0)

    def _whole(c, j):
        return (0, 0)

    out_pad = pl.pallas_call(
        _fused_kernel,
        out_shape=jax.ShapeDtypeStruct((B_pad, NF), jnp.float32),
        grid=(2, n_inner),
        in_specs=[
            pl.BlockSpec((3, TB, D), _tile3),
            pl.BlockSpec((2 * D, D), _whole),
            pl.BlockSpec((D, D), _whole),
            pl.BlockSpec((D, D), _whole),
            pl.BlockSpec((2 * D, D), _whole),
            pl.BlockSpec((3 * D, NF), _whole),
            pl.BlockSpec((1, NF), _whole),
        ],
        out_specs=pl.BlockSpec((TB, NF), _tile),
        compiler_params=compiler_params,
    )(ft, w01, w02a, w02b, w12, wf, bias2)

    return out_pad[:B, :6].astype(features.dtype)


# speedup vs baseline: 1.0187x; 1.0187x over previous
import jax
import jax.numpy as jnp
from jax.experimental import pallas as pl
from jax.experimental.pallas import tpu as pltpu


def _round_up(x, m):
    return (x + m - 1) // m * m


def _fused_kernel(ft_ref, w01_ref, w02a_ref, w02b_ref, w12_ref,
                  wf_ref, bd_ref, bf_ref, out_ref, bias_s):
    D = w02a_ref.shape[0]
    wf = wf_ref[...]

    @pl.when(pl.program_id(1) == 0)
    def _prep_bias():
        bias_s[...] = (jnp.dot(bd_ref[...], wf,
                               preferred_element_type=jnp.float32)
                       + bf_ref[...])

    f0 = ft_ref[0]
    f1 = ft_ref[1]
    f2 = ft_ref[2]

    r01 = jnp.dot(f0, w01_ref[:D], preferred_element_type=jnp.float32)
    r01 = r01 + jnp.dot(f1, w01_ref[D:], preferred_element_type=jnp.float32)
    r12 = jnp.dot(f1, w12_ref[:D], preferred_element_type=jnp.float32)
    r12 = r12 + jnp.dot(f2, w12_ref[D:], preferred_element_type=jnp.float32)
    r02 = jnp.dot(f0, w02a_ref[...], preferred_element_type=jnp.float32)
    r02 = r02 + jnp.dot(f2, w02b_ref[...],
                        preferred_element_type=jnp.float32)

    out = jnp.dot(r01, wf[:D], preferred_element_type=jnp.float32)
    out = out + jnp.dot(r02, wf[D:2 * D], preferred_element_type=jnp.float32)
    out = out + jnp.dot(r12, wf[2 * D:], preferred_element_type=jnp.float32)
    out = out + bias_s[...]
    out_ref[...] = out[:, :out_ref.shape[1]]


def kernel(features, wd, bd, wf, bf):
    B, three, D = features.shape
    NF = wf.shape[1]

    w01 = wd[:2 * D, :D].astype(jnp.bfloat16)
    w12 = wd[D:, 2 * D:].astype(jnp.bfloat16)
    w02a = wd[:D, D:2 * D].astype(jnp.bfloat16)
    w02b = wd[2 * D:, D:2 * D].astype(jnp.bfloat16)

    ft = features.transpose((1, 0, 2)).astype(jnp.bfloat16)

    TB = 1024
    B_pad = _round_up(B, 2 * TB)
    if B_pad != B:
        ft = jnp.pad(ft, ((0, 0), (0, B_pad - B), (0, 0)))
    n_inner = B_pad // TB // 2

    compiler_params = pltpu.CompilerParams(
        dimension_semantics=("parallel", "arbitrary"),
        vmem_limit_bytes=64 * 1024 * 1024,
    )

    def _tile3(c, j, n=n_inner):
        return (0, c * n + j, 0)

    def _tile(c, j, n=n_inner):
        return (c * n + j, 0)

    def _whole(c, j):
        return (0, 0)

    out_pad = pl.pallas_call(
        _fused_kernel,
        out_shape=jax.ShapeDtypeStruct((B_pad, 6), jnp.float32),
        grid=(2, n_inner),
        in_specs=[
            pl.BlockSpec((3, TB, D), _tile3),
            pl.BlockSpec((2 * D, D), _whole),
            pl.BlockSpec((D, D), _whole),
            pl.BlockSpec((D, D), _whole),
            pl.BlockSpec((2 * D, D), _whole),
            pl.BlockSpec((3 * D, NF), _whole),
            pl.BlockSpec((1, 3 * D), _whole),
            pl.BlockSpec((1, NF), _whole),
        ],
        out_specs=pl.BlockSpec((TB, 6), _tile),
        scratch_shapes=[pltpu.VMEM((1, NF), jnp.float32)],
        compiler_params=compiler_params,
    )(ft, w01, w02a, w02b, w12, wf, bd, bf)

    if B_pad != B:
        out_pad = out_pad[:B]
    return out_pad.astype(features.dtype)
